# baseline (speedup 1.0000x reference)
import jax
import jax.numpy as jnp
from jax import lax
from jax.experimental import pallas as pl
from jax.experimental.pallas import tpu as pltpu
from jax.experimental.pallas import tpu_sc as plsc

NUM_KEEP = 4096

_NC = 2
_NS = 16
_NW = _NC * _NS

_CHUNK = 32


_NBUF = 3


def _build_gather(rows_total: int, feat: int):
    rpw = rows_total // _NW
    nch = rpw // _CHUNK
    mesh = plsc.VectorSubcoreMesh(core_axis_name="c", subcore_axis_name="s")

    @pl.kernel(
        mesh=mesh,
        out_type=jax.ShapeDtypeStruct((rows_total, feat), jnp.float32),
        scratch_types=(
            [pltpu.VMEM((rpw,), jnp.int32)]
            + [pltpu.VMEM((_CHUNK, feat), jnp.float32)] * _NBUF
            + [pltpu.SemaphoreType.DMA] * (2 * _NBUF)
        ),
    )
    def gather_rows(table_hbm, idx_hbm, out_hbm, idx_v, *rest):
        bufs = rest[:_NBUF]
        in_sems = rest[_NBUF:2 * _NBUF]
        out_sems = rest[2 * _NBUF:]
        wid = lax.axis_index("s") * _NC + lax.axis_index("c")
        base = wid * rpw
        pltpu.sync_copy(idx_hbm.at[pl.ds(base, rpw)], idx_v)

        def start_in(c):
            s = c % _NBUF
            return pltpu.async_copy(
                table_hbm.at[idx_v.at[pl.ds(c * _CHUNK, _CHUNK)]],
                bufs[s], in_sems[s])

        def start_out(c):
            s = c % _NBUF
            return pltpu.async_copy(
                bufs[s], out_hbm.at[pl.ds(base + c * _CHUNK, _CHUNK)],
                out_sems[s])

        pending_in = [None] * _NBUF
        pending_out = [None] * _NBUF
        for c in range(min(2, nch)):
            pending_in[c % _NBUF] = start_in(c)
        for c in range(nch):
            s = c % _NBUF
            pending_in[s].wait()
            pending_out[s] = start_out(c)
            nxt = c + 2
            if nxt < nch:
                if c >= 1:
                    pending_out[(c - 1) % _NBUF].wait()
                pending_in[nxt % _NBUF] = start_in(nxt)
        for c in range(max(nch - _NBUF, 0), nch):
            if pending_out[c % _NBUF] is not None:
                pending_out[c % _NBUF].wait()

    return gather_rows


def kernel(x):
    b, t, f = x.shape
    keep = min(t, NUM_KEEP)
    skey = jax.random.key(42)
    scores = jax.random.uniform(skey, (b, t), dtype=jnp.float32)
    idx = jnp.argsort(scores, axis=1)[:, :keep]
    flat_idx = (idx.astype(jnp.int32)
                + jnp.arange(b, dtype=jnp.int32)[:, None] * t).reshape(-1)
    table = x.reshape(b * t, f)
    out = _build_gather(b * keep, f)(table, flat_idx)
    return out.reshape(b, keep, f)
